# baseline (speedup 1.0000x reference)
import jax
import jax.numpy as jnp
from jax import lax
from jax.experimental import pallas as pl
from jax.experimental.pallas import tpu as pltpu

_BT = 256


def _router_body(x_ref, wt_ref, idx_ref, w_ref, counts_ref, psum_ref,
                 lb_ref, z_ref, bal_ref, ent_ref):
    i = pl.program_id(0)
    nsteps = pl.num_programs(0)
    bt = x_ref.shape[0]
    e = wt_ref.shape[1]
    t_total = bt * nsteps
    k_top = idx_ref.shape[0]

    logits = jnp.dot(x_ref[...], wt_ref[...],
                     preferred_element_type=jnp.float32)
    lt = logits.T

    m = jnp.max(lt, axis=0, keepdims=True)
    eu = jnp.exp(lt - m)
    s = jnp.sum(eu, axis=0, keepdims=True)
    probs = eu / s

    lse = m + jnp.log(s)
    z_blk = jnp.sum(lse * lse)
    ent_blk = -jnp.sum(probs * jnp.log(probs + 1e-10))
    psum_blk = jnp.sum(probs, axis=1, keepdims=True)

    iota_e = lax.broadcasted_iota(jnp.int32, (e, bt), 0)
    key = (lax.bitcast_convert_type(probs, jnp.int32) & ~63) | (e - 1 - iota_e)
    cnt = jnp.zeros((e, bt), jnp.int32)
    idx_rows = []
    w_rows = []
    for _ in range(k_top):
        kmax = jnp.max(key, axis=0, keepdims=True)
        idxk = (e - 1) - (kmax & 63)
        wk = lax.bitcast_convert_type(kmax & ~63, jnp.float32)
        onehot = iota_e == idxk
        cnt = cnt + onehot.astype(jnp.int32)
        idx_rows.append(idxk)
        w_rows.append(wk)
        key = jnp.where(onehot, jnp.int32(-1), key)

    w_all = jnp.concatenate(w_rows, axis=0)
    wsum = jnp.sum(w_all, axis=0, keepdims=True)
    idx_ref[...] = jnp.concatenate(idx_rows, axis=0)
    w_ref[...] = w_all / wsum

    counts_blk = jnp.sum(cnt.astype(jnp.float32), axis=1,
                         keepdims=True)

    @pl.when(i == 0)
    def _init():
        counts_ref[...] = jnp.zeros_like(counts_ref)
        psum_ref[...] = jnp.zeros_like(psum_ref)
        z_ref[0, 0] = 0.0
        ent_ref[0, 0] = 0.0
        lb_ref[0, 0] = 0.0
        bal_ref[0, 0] = 0.0

    counts_ref[...] += counts_blk
    psum_ref[...] += psum_blk
    z_ref[0, 0] += z_blk
    ent_ref[0, 0] += ent_blk

    @pl.when(i == nsteps - 1)
    def _finalize():
        counts_f = counts_ref[...]
        frac = counts_f / (t_total * k_top)
        meanp = psum_ref[...] / t_total
        psum_ref[...] = meanp
        lb_ref[0, 0] = e * jnp.sum(frac * meanp)
        bal_ref[0, 0] = jnp.max(frac) * e
        z_ref[0, 0] = z_ref[0, 0] / t_total
        ent_ref[0, 0] = ent_ref[0, 0] / t_total


def kernel(hidden_states, W):
    b, s, h = hidden_states.shape
    e = W.shape[0]
    k_top = 8
    t = b * s
    bt = _BT
    grid = (t // bt,)

    x2 = hidden_states.reshape(t, h)
    wt = W.T

    smem_scalar = pl.BlockSpec((1, 1), lambda i: (0, 0),
                               memory_space=pltpu.SMEM)
    out_shapes = (
        jax.ShapeDtypeStruct((k_top, t), jnp.int32),
        jax.ShapeDtypeStruct((k_top, t), jnp.float32),
        jax.ShapeDtypeStruct((e, 1), jnp.float32),
        jax.ShapeDtypeStruct((e, 1), jnp.float32),
        jax.ShapeDtypeStruct((1, 1), jnp.float32),
        jax.ShapeDtypeStruct((1, 1), jnp.float32),
        jax.ShapeDtypeStruct((1, 1), jnp.float32),
        jax.ShapeDtypeStruct((1, 1), jnp.float32),
    )
    out_specs = (
        pl.BlockSpec((k_top, bt), lambda i: (0, i)),
        pl.BlockSpec((k_top, bt), lambda i: (0, i)),
        pl.BlockSpec((e, 1), lambda i: (0, 0)),
        pl.BlockSpec((e, 1), lambda i: (0, 0)),
        smem_scalar, smem_scalar, smem_scalar, smem_scalar,
    )
    in_specs = (
        pl.BlockSpec((bt, h), lambda i: (i, 0)),
        pl.BlockSpec((h, e), lambda i: (0, 0)),
    )

    idx, w, counts, meanp, lb, z, bal, ent = pl.pallas_call(
        _router_body,
        grid=grid,
        in_specs=in_specs,
        out_specs=out_specs,
        out_shape=out_shapes,
        compiler_params=pltpu.CompilerParams(
            dimension_semantics=("arbitrary",)),
    )(x2, wt)

    return (idx.T.reshape(b, s, k_top), w.T.reshape(b, s, k_top),
            lb[0, 0], z[0, 0], bal[0, 0], ent[0, 0],
            counts.reshape(e), meanp.reshape(e))


# speedup vs baseline: 1.2200x; 1.2200x over previous
import jax
import jax.numpy as jnp
from jax import lax
from jax.experimental import pallas as pl
from jax.experimental.pallas import tpu as pltpu

_BT = 1024


def _router_body(x_ref, wt_ref, idx_ref, w_ref, counts_ref, psum_ref,
                 lb_ref, z_ref, bal_ref, ent_ref):
    i = pl.program_id(0)
    nsteps = pl.num_programs(0)
    bt = x_ref.shape[0]
    e = wt_ref.shape[1]
    t_total = bt * nsteps
    k_top = idx_ref.shape[0]

    logits = jnp.dot(x_ref[...], wt_ref[...],
                     preferred_element_type=jnp.float32)
    lt = logits.T

    m = jnp.max(lt, axis=0, keepdims=True)
    eu = jnp.exp(lt - m)
    s = jnp.sum(eu, axis=0, keepdims=True)
    probs = eu / s

    lse = m + jnp.log(s)
    z_blk = jnp.sum(lse * lse)
    ent_blk = -jnp.sum(probs * jnp.log(probs + 1e-10))
    psum_blk = jnp.sum(probs, axis=1, keepdims=True)

    iota_e = lax.broadcasted_iota(jnp.int32, (e, bt), 0)
    key = (lax.bitcast_convert_type(probs, jnp.int32) & ~63) | (e - 1 - iota_e)
    cnt = jnp.zeros((e, bt), jnp.int32)
    idx_rows = []
    w_rows = []
    for _ in range(k_top):
        kmax = jnp.max(key, axis=0, keepdims=True)
        idxk = (e - 1) - (kmax & 63)
        wk = lax.bitcast_convert_type(kmax & ~63, jnp.float32)
        onehot = iota_e == idxk
        cnt = cnt + onehot.astype(jnp.int32)
        idx_rows.append(idxk)
        w_rows.append(wk)
        key = jnp.where(onehot, jnp.int32(-1), key)

    w_all = jnp.concatenate(w_rows, axis=0)
    wsum = jnp.sum(w_all, axis=0, keepdims=True)
    idx_ref[...] = jnp.concatenate(idx_rows, axis=0)
    w_ref[...] = w_all / wsum

    counts_blk = jnp.sum(cnt.astype(jnp.float32), axis=1,
                         keepdims=True)

    @pl.when(i == 0)
    def _init():
        counts_ref[...] = jnp.zeros_like(counts_ref)
        psum_ref[...] = jnp.zeros_like(psum_ref)
        z_ref[0, 0] = 0.0
        ent_ref[0, 0] = 0.0
        lb_ref[0, 0] = 0.0
        bal_ref[0, 0] = 0.0

    counts_ref[...] += counts_blk
    psum_ref[...] += psum_blk
    z_ref[0, 0] += z_blk
    ent_ref[0, 0] += ent_blk

    @pl.when(i == nsteps - 1)
    def _finalize():
        counts_f = counts_ref[...]
        frac = counts_f / (t_total * k_top)
        meanp = psum_ref[...] / t_total
        psum_ref[...] = meanp
        lb_ref[0, 0] = e * jnp.sum(frac * meanp)
        bal_ref[0, 0] = jnp.max(frac) * e
        z_ref[0, 0] = z_ref[0, 0] / t_total
        ent_ref[0, 0] = ent_ref[0, 0] / t_total


def kernel(hidden_states, W):
    b, s, h = hidden_states.shape
    e = W.shape[0]
    k_top = 8
    t = b * s
    bt = _BT
    grid = (t // bt,)

    x2 = hidden_states.reshape(t, h)
    wt = W.T

    smem_scalar = pl.BlockSpec((1, 1), lambda i: (0, 0),
                               memory_space=pltpu.SMEM)
    out_shapes = (
        jax.ShapeDtypeStruct((k_top, t), jnp.int32),
        jax.ShapeDtypeStruct((k_top, t), jnp.float32),
        jax.ShapeDtypeStruct((e, 1), jnp.float32),
        jax.ShapeDtypeStruct((e, 1), jnp.float32),
        jax.ShapeDtypeStruct((1, 1), jnp.float32),
        jax.ShapeDtypeStruct((1, 1), jnp.float32),
        jax.ShapeDtypeStruct((1, 1), jnp.float32),
        jax.ShapeDtypeStruct((1, 1), jnp.float32),
    )
    out_specs = (
        pl.BlockSpec((k_top, bt), lambda i: (0, i)),
        pl.BlockSpec((k_top, bt), lambda i: (0, i)),
        pl.BlockSpec((e, 1), lambda i: (0, 0)),
        pl.BlockSpec((e, 1), lambda i: (0, 0)),
        smem_scalar, smem_scalar, smem_scalar, smem_scalar,
    )
    in_specs = (
        pl.BlockSpec((bt, h), lambda i: (i, 0)),
        pl.BlockSpec((h, e), lambda i: (0, 0)),
    )

    idx, w, counts, meanp, lb, z, bal, ent = pl.pallas_call(
        _router_body,
        grid=grid,
        in_specs=in_specs,
        out_specs=out_specs,
        out_shape=out_shapes,
        compiler_params=pltpu.CompilerParams(
            dimension_semantics=("arbitrary",)),
    )(x2, wt)

    return (idx.T.reshape(b, s, k_top), w.T.reshape(b, s, k_top),
            lb[0, 0], z[0, 0], bal[0, 0], ent[0, 0],
            counts.reshape(e), meanp.reshape(e))
